# Initial kernel scaffold; baseline (speedup 1.0000x reference)
import jax
import jax.numpy as jnp
from jax.experimental import pallas as pl


def kernel(x, embedding):
    raise NotImplementedError("write your pallas kernel here")


# speedup vs baseline: 1.6026x; 1.6026x over previous
import functools
import math

import jax
import jax.numpy as jnp
from jax import lax
from jax.experimental import pallas as pl
from jax.experimental.pallas import tpu as pltpu
from jax.experimental.pallas import tpu_sc as plsc

_EMBDIM = 32
_SCALE = math.sqrt(_EMBDIM)
_CHUNK = 1024
_UNROLL = 8


@functools.partial(jax.jit, static_argnames=("chunk",))
def _gather_dup(x_flat, table, chunk=_CHUNK):
    b_total = x_flat.shape[0]
    info = plsc.get_sparse_core_info()
    num_cores, num_subcores = info.num_cores, info.num_subcores
    num_workers = num_cores * num_subcores
    b_per_w = b_total // num_workers
    n_chunks = b_per_w // chunk
    assert b_per_w * num_workers == b_total and n_chunks * chunk == b_per_w

    mesh = plsc.VectorSubcoreMesh(core_axis_name="c", subcore_axis_name="s")

    @functools.partial(
        pl.kernel,
        out_type=jax.ShapeDtypeStruct((b_total, 2, _EMBDIM), jnp.float32),
        mesh=mesh,
        scratch_types=[
            pltpu.VMEM((chunk,), jnp.int32),
            pltpu.VMEM((chunk, _EMBDIM), jnp.float32),
            pltpu.SemaphoreType.DMA,
        ],
        compiler_params=pltpu.CompilerParams(use_tc_tiling_on_sc=False),
    )
    def k(x_hbm, tab_hbm, out_hbm, idx_v, rows_v, sem):
        wid = lax.axis_index("s") * num_cores + lax.axis_index("c")
        base = wid * b_per_w

        def do_chunk(i, carry):
            cbase = base + i * chunk
            pltpu.sync_copy(x_hbm.at[pl.ds(cbase, chunk)], idx_v)
            pltpu.async_copy(tab_hbm.at[idx_v], rows_v, sem).wait()

            def scale_rows(j, c):
                r0 = j * _UNROLL
                for u in range(_UNROLL):
                    rows_v[r0 + u, pl.ds(0, 16)] = (
                        rows_v[r0 + u, pl.ds(0, 16)] * _SCALE)
                    rows_v[r0 + u, pl.ds(16, 16)] = (
                        rows_v[r0 + u, pl.ds(16, 16)] * _SCALE)
                return c

            lax.fori_loop(0, chunk // _UNROLL, scale_rows, 0)

            pltpu.sync_copy(rows_v, out_hbm.at[pl.ds(cbase, chunk), 0, :])
            pltpu.sync_copy(rows_v, out_hbm.at[pl.ds(cbase, chunk), 1, :])
            return carry

        lax.fori_loop(0, n_chunks, do_chunk, 0)

    return k(x_flat, table)


def kernel(x, embedding):
    x_flat = x.reshape(-1).astype(jnp.int32)
    out = _gather_dup(x_flat, embedding)
    return out.reshape(*x.shape, 2 * _EMBDIM)
